# baseline (speedup 1.0000x reference)
import functools

import jax
import jax.numpy as jnp
from jax import lax
from jax.experimental import pallas as pl
from jax.experimental.pallas import tpu as pltpu
from jax.experimental.pallas import tpu_sc as plsc

NC, NS = 2, 16
NW = NC * NS
N = 10000
H = 128
E = 320000
L = 100000
CH = 128
NCHUNK = E // CH
SEG_STEPS = 79
LP = 100096
LCHUNK = LP // CH
DEC_STEPS = 13
RC = 400
NRC = N // RC
K_RC = -(-NRC // NS)

_MESH = plsc.VectorSubcoreMesh(
    core_axis_name="c", subcore_axis_name="s", num_cores=NC, num_subcores=NS)


def _seg_pair_body(edg_a, tbl_a, init_a, edg_b, tbl_b, init_b,
                   out_a, out_b, *scr):
    sidx = scr[0:2]
    didx = scr[2:4]
    rows = scr[4:6]
    acc = scr[6]
    semg = scr[7:9]
    c = lax.axis_index("c")
    t = lax.axis_index("s")

    def rows_loop(body_fn):
        def body(k, carry):
            cid = k * NS + t

            @pl.when(cid < NRC)
            def _():
                body_fn(cid * RC)

            return carry

        lax.fori_loop(0, K_RC, body, 0)

    def run(edg_h, tbl_h, init_h, out_h):
        def prep(i, j):
            cid = i * NS + t

            @pl.when(cid < NCHUNK)
            def _():
                base = cid * CH
                pltpu.sync_copy(edg_h.at[pl.ds(base, CH)], sidx[j])
                pltpu.sync_copy(edg_h.at[pl.ds(E + base, CH)], didx[j])
                pltpu.async_copy(tbl_h.at[sidx[j]], rows[j], semg[j])

        def drain(i, j):
            cid = i * NS + t

            @pl.when(cid < NCHUNK)
            def _():
                pltpu.make_async_copy(
                    tbl_h.at[sidx[j]], rows[j], semg[j]).wait()
                pltpu.sync_copy(rows[j], acc.at[didx[j]], add=True)

        rows_loop(lambda r0: pltpu.sync_copy(
            init_h.at[pl.ds(r0, RC)], acc.at[pl.ds(r0, RC)]))
        prep(0, 0)
        plsc.subcore_barrier()

        def pair(kk, carry):
            i0 = 2 * kk
            prep(i0 + 1, 1)
            drain(i0, 0)
            prep(i0 + 2, 0)
            drain(i0 + 1, 1)
            return carry

        lax.fori_loop(0, SEG_STEPS, pair, 0)
        plsc.subcore_barrier()
        rows_loop(lambda r0: pltpu.sync_copy(
            acc.at[pl.ds(r0, RC)], out_h.at[pl.ds(r0, RC)]))

    @pl.when(c == 0)
    def _():
        run(edg_a, tbl_a, init_a, out_a)

    @pl.when(c == 1)
    def _():
        run(edg_b, tbl_b, init_b, out_b)


_seg_pair = pl.kernel(
    _seg_pair_body,
    out_type=(jax.ShapeDtypeStruct((N, H), jnp.float32),
              jax.ShapeDtypeStruct((N, H), jnp.float32)),
    mesh=_MESH,
    scratch_types=(
        [pltpu.VMEM((CH,), jnp.int32)] * 4
        + [pltpu.VMEM((CH, H), jnp.float32)] * 2
        + [pltpu.VMEM_SHARED((N, H), jnp.float32)]
        + [pltpu.SemaphoreType.DMA] * 2
    ),
)


def _dec_gather_body(ud, up, lbl_h, wb_h, out_h, *scr):
    ridx = scr[0:2]
    cidx = scr[2:4]
    buf = scr[4:6]
    outb = scr[6:8]
    wb_v = scr[8]
    sem_a = scr[9:11]
    sem_b = scr[11:13]
    wid = lax.axis_index("c") * NS + lax.axis_index("s")
    rids = [lax.iota(jnp.int32, 16) + (g * 16) for g in range(8)]

    pltpu.sync_copy(wb_h, wb_v)

    def prep(i, j):
        cid = i * NW + wid

        @pl.when(cid < LCHUNK)
        def _():
            pltpu.sync_copy(lbl_h.at[pl.ds(cid * CH, CH)], ridx[j])
            pltpu.async_copy(ud.at[ridx[j]], buf[j], sem_a[j])

    def mid(i, j):
        cid = i * NW + wid

        @pl.when(cid < LCHUNK)
        def _():
            pltpu.make_async_copy(ud.at[ridx[j]], buf[j], sem_a[j]).wait()
            pltpu.sync_copy(lbl_h.at[pl.ds(L + cid * CH, CH)], cidx[j])
            pltpu.async_copy(up.at[cidx[j]], buf[j], sem_b[j], add=True)

    def fin(i, j):
        cid = i * NW + wid

        @pl.when(cid < LCHUNK)
        def _():
            pltpu.make_async_copy(up.at[cidx[j]], buf[j], sem_b[j]).wait()

            def fpair(fk, accs):
                new = accs
                for fo in range(2):
                    f = 2 * fk + fo
                    fv = jnp.full((16,), f, jnp.int32)
                    wv = plsc.load_gather(wb_v, [fv])
                    new = tuple(
                        new[g] + jnp.maximum(
                            plsc.load_gather(buf[j], [rids[g], fv]), 0.0) * wv
                        for g in range(8))
                return new

            b2v = wb_v[pl.ds(H, 16)]
            accs = lax.fori_loop(0, H // 2, fpair, (b2v,) * 8)
            for g in range(8):
                outb[j][pl.ds(g * 16, 16)] = accs[g]
            pltpu.sync_copy(outb[j], out_h.at[pl.ds(cid * CH, CH)])

    prep(0, 0)

    def pair(kk, carry):
        i0 = 2 * kk
        prep(i0 + 1, 1)
        mid(i0, 0)
        fin(i0, 0)
        prep(i0 + 2, 0)
        mid(i0 + 1, 1)
        fin(i0 + 1, 1)
        return carry

    lax.fori_loop(0, DEC_STEPS, pair, 0)


_dec_gather = pl.kernel(
    _dec_gather_body,
    out_type=jax.ShapeDtypeStruct((LP,), jnp.float32),
    mesh=_MESH,
    compiler_params=pltpu.CompilerParams(needs_layout_passes=False),
    scratch_types=(
        [pltpu.VMEM((CH,), jnp.int32)] * 4
        + [pltpu.VMEM((CH, H), jnp.float32)] * 2
        + [pltpu.VMEM((CH,), jnp.float32)] * 2
        + [pltpu.VMEM((H + 16,), jnp.float32)]
        + [pltpu.SemaphoreType.DMA] * 4
    ),
)


def _quad_body(a_ref, b_ref, w1, w2, w3, w4, bias1, bias2,
               o1, o2, o3, o4, *, relu):
    a = a_ref[...]
    b = b_ref[...]
    if relu:
        a = jnp.maximum(a, 0.0)
        b = jnp.maximum(b, 0.0)
    f32 = jnp.float32
    o1[...] = jnp.dot(a, w1[...], preferred_element_type=f32)
    o2[...] = jnp.dot(b, w2[...], preferred_element_type=f32) + bias1[...]
    o3[...] = jnp.dot(b, w3[...], preferred_element_type=f32)
    o4[...] = jnp.dot(a, w4[...], preferred_element_type=f32) + bias2[...]


def _make_quad(relu):
    blk = 1000
    grid = N // blk
    row_spec = pl.BlockSpec((blk, H), lambda i: (i, 0))
    full_spec = pl.BlockSpec((H, H), lambda i: (0, 0))
    bias_spec = pl.BlockSpec((1, H), lambda i: (0, 0))
    return pl.pallas_call(
        functools.partial(_quad_body, relu=relu),
        grid=(grid,),
        in_specs=[row_spec, row_spec, full_spec, full_spec, full_spec,
                  full_spec, bias_spec, bias_spec],
        out_specs=[row_spec, row_spec, row_spec, row_spec],
        out_shape=[jax.ShapeDtypeStruct((N, H), jnp.float32)] * 4,
    )


_quad_plain = _make_quad(relu=False)
_quad_relu = _make_quad(relu=True)


def _dual_body(a_ref, b_ref, w1, w2, bias1, o1, o2):
    f32 = jnp.float32
    o1[...] = jnp.dot(a_ref[...], w1[...], preferred_element_type=f32) + bias1[...]
    o2[...] = jnp.dot(b_ref[...], w2[...], preferred_element_type=f32)


def _make_dual():
    blk = 1000
    grid = N // blk
    row_spec = pl.BlockSpec((blk, H), lambda i: (i, 0))
    full_spec = pl.BlockSpec((H, H), lambda i: (0, 0))
    bias_spec = pl.BlockSpec((1, H), lambda i: (0, 0))
    return pl.pallas_call(
        _dual_body,
        grid=(grid,),
        in_specs=[row_spec, row_spec, full_spec, full_spec, bias_spec],
        out_specs=[row_spec, row_spec],
        out_shape=[jax.ShapeDtypeStruct((N, H), jnp.float32)] * 2,
    )


_dual = _make_dual()


def kernel(x_drug, x_protein, edge_index_drug_protein, edge_index_protein_drug,
           edge_label_index, Wl1_dp, bl1_dp, Wr1_dp, Wl1_pd, bl1_pd, Wr1_pd,
           Wl2_dp, bl2_dp, Wr2_dp, Wl2_pd, bl2_pd, Wr2_pd,
           Wdec1, bdec1, Wdec2, bdec2):
    edg_dp = edge_index_drug_protein.reshape(-1)
    edg_pd = edge_index_protein_drug.reshape(-1)
    lbl = jnp.concatenate(
        [edge_label_index.reshape(-1),
         jnp.zeros((LP - L,), edge_label_index.dtype)])

    b1 = bl1_dp.reshape(1, H)
    b2 = bl1_pd.reshape(1, H)
    b3 = bl2_dp.reshape(1, H)
    b4 = bl2_pd.reshape(1, H)

    tbl_dp, init_prot, tbl_pd, init_drug = _quad_plain(
        x_drug, x_protein, Wl1_dp, Wr1_dp, Wl1_pd, Wr1_pd, b1, b2)
    hpre_prot, hpre_drug = _seg_pair(
        edg_dp, tbl_dp, init_prot, edg_pd, tbl_pd, init_drug)

    tbl2_dp, init2_prot, tbl2_pd, init2_drug = _quad_relu(
        hpre_drug, hpre_prot, Wl2_dp, Wr2_dp, Wl2_pd, Wr2_pd, b3, b4)
    z_prot, z_drug = _seg_pair(
        edg_dp, tbl2_dp, init2_prot, edg_pd, tbl2_pd, init2_drug)

    u_drug, u_prot = _dual(z_drug, z_prot, Wdec1[:H], Wdec1[H:],
                           bdec1.reshape(1, H))
    wb = jnp.concatenate([Wdec2.reshape(-1), jnp.broadcast_to(bdec2, (16,))])
    out = _dec_gather(u_drug, u_prot, lbl, wb)[:L]
    return (z_drug, z_prot, out)


# speedup vs baseline: 1.4622x; 1.4622x over previous
import functools

import jax
import jax.numpy as jnp
from jax import lax
from jax.experimental import pallas as pl
from jax.experimental.pallas import tpu as pltpu
from jax.experimental.pallas import tpu_sc as plsc

NC, NS = 2, 16
NW = NC * NS
N = 10000
H = 128
E = 320000
L = 100000
CH = 128
NCHUNK = E // CH
NGRP = NCHUNK // 4
SEG_STEPS = 20
LP = 100096
LCHUNK = LP // CH
DEC_STEPS = 13
RC = 400
NRC = N // RC
K_RC = -(-NRC // NS)

_MESH = plsc.VectorSubcoreMesh(
    core_axis_name="c", subcore_axis_name="s", num_cores=NC, num_subcores=NS)


def _seg_pair_body(idx_a, tbl_a, init_a, idx_b, tbl_b, init_b,
                   out_a, out_b, *scr):
    rows = scr[0:2]
    ib = scr[2:4]
    acc = scr[4]
    semg = scr[5:7]
    semi = scr[7:9]
    c = lax.axis_index("c")
    t = lax.axis_index("s")

    def rows_loop(body_fn):
        def body(k, carry):
            cid = k * NS + t

            @pl.when(cid < NRC)
            def _():
                body_fn(cid * RC)

            return carry

        lax.fori_loop(0, K_RC, body, 0)

    def run(idx_h, tbl_h, init_h, out_h):
        def pfx(k, jg):
            gid = k * NS + t

            @pl.when(gid < NGRP)
            def _():
                pltpu.async_copy(idx_h.at[gid], ib[jg], semi[jg])

        def wait_i(k, jg):
            gid = k * NS + t

            @pl.when(gid < NGRP)
            def _():
                pltpu.make_async_copy(
                    idx_h.at[gid], ib[jg], semi[jg]).wait()

        def g(k, m, j, jg):
            cid = 4 * (k * NS + t) + m

            @pl.when(cid < NCHUNK)
            def _():
                pltpu.async_copy(tbl_h.at[ib[jg].at[m]], rows[j], semg[j])

        def d(k, m, j, jg, check_neg=False):
            cid = 4 * (k * NS + t) + m
            cond = cid < NCHUNK if not check_neg else (
                (cid >= 0) & (cid < NCHUNK))

            @pl.when(cond)
            def _():
                pltpu.make_async_copy(
                    tbl_h.at[ib[jg].at[m]], rows[j], semg[j]).wait()
                pltpu.sync_copy(rows[j], acc.at[ib[jg].at[4 + m]], add=True)

        pfx(0, 0)
        rows_loop(lambda r0: pltpu.sync_copy(
            init_h.at[pl.ds(r0, RC)], acc.at[pl.ds(r0, RC)]))
        plsc.subcore_barrier()

        def two_groups(gg, carry):
            k0 = 2 * gg
            k1 = k0 + 1
            wait_i(k0, 0)
            g(k0, 0, 0, 0)
            d(k1 - 2, 3, 1, 1, check_neg=True)
            pfx(k1, 1)
            g(k0, 1, 1, 0)
            d(k0, 0, 0, 0)
            g(k0, 2, 0, 0)
            d(k0, 1, 1, 0)
            g(k0, 3, 1, 0)
            d(k0, 2, 0, 0)
            wait_i(k1, 1)
            g(k1, 0, 0, 1)
            d(k0, 3, 1, 0)
            pfx(k0 + 2, 0)
            g(k1, 1, 1, 1)
            d(k1, 0, 0, 1)
            g(k1, 2, 0, 1)
            d(k1, 1, 1, 1)
            g(k1, 3, 1, 1)
            d(k1, 2, 0, 1)
            return carry

        lax.fori_loop(0, SEG_STEPS, two_groups, 0)
        d(2 * SEG_STEPS - 1, 3, 1, 1)
        plsc.subcore_barrier()
        rows_loop(lambda r0: pltpu.sync_copy(
            acc.at[pl.ds(r0, RC)], out_h.at[pl.ds(r0, RC)]))

    @pl.when(c == 0)
    def _():
        run(idx_a, tbl_a, init_a, out_a)

    @pl.when(c == 1)
    def _():
        run(idx_b, tbl_b, init_b, out_b)


_seg_pair = pl.kernel(
    _seg_pair_body,
    out_type=(jax.ShapeDtypeStruct((N, H), jnp.float32),
              jax.ShapeDtypeStruct((N, H), jnp.float32)),
    mesh=_MESH,
    scratch_types=(
        [pltpu.VMEM((CH, H), jnp.float32)] * 2
        + [pltpu.VMEM((8, CH), jnp.int32)] * 2
        + [pltpu.VMEM_SHARED((N, H), jnp.float32)]
        + [pltpu.SemaphoreType.DMA] * 4
    ),
)


def _dec_gather_body(ud, up, lbl_h, s_h, *scr):
    ridx = scr[0:2]
    cidx = scr[2:4]
    buf = scr[4:6]
    sem_a = scr[6:8]
    sem_b = scr[8:10]
    wid = lax.axis_index("c") * NS + lax.axis_index("s")

    def prep(i, j):
        cid = i * NW + wid

        @pl.when(cid < LCHUNK)
        def _():
            pltpu.sync_copy(lbl_h.at[pl.ds(cid * CH, CH)], ridx[j])
            pltpu.async_copy(ud.at[ridx[j]], buf[j], sem_a[j])

    def mid(i, j):
        cid = i * NW + wid

        @pl.when(cid < LCHUNK)
        def _():
            pltpu.make_async_copy(ud.at[ridx[j]], buf[j], sem_a[j]).wait()
            pltpu.sync_copy(lbl_h.at[pl.ds(L + cid * CH, CH)], cidx[j])
            pltpu.async_copy(up.at[cidx[j]], buf[j], sem_b[j], add=True)

    def fin(i, j):
        cid = i * NW + wid

        @pl.when(cid < LCHUNK)
        def _():
            pltpu.make_async_copy(up.at[cidx[j]], buf[j], sem_b[j]).wait()
            pltpu.sync_copy(buf[j], s_h.at[pl.ds(cid * CH, CH)])

    prep(0, 0)

    def pair(kk, carry):
        i0 = 2 * kk
        prep(i0 + 1, 1)
        mid(i0, 0)
        fin(i0, 0)
        prep(i0 + 2, 0)
        mid(i0 + 1, 1)
        fin(i0 + 1, 1)
        return carry

    lax.fori_loop(0, DEC_STEPS, pair, 0)


_dec_gather = pl.kernel(
    _dec_gather_body,
    out_type=jax.ShapeDtypeStruct((LP, H), jnp.float32),
    mesh=_MESH,
    scratch_types=(
        [pltpu.VMEM((CH,), jnp.int32)] * 4
        + [pltpu.VMEM((CH, H), jnp.float32)] * 2
        + [pltpu.SemaphoreType.DMA] * 4
    ),
)


def _dec_out_body(s_ref, w2_ref, b2_ref, o_ref):
    s = jnp.maximum(s_ref[...], 0.0)
    o_ref[...] = jnp.sum(s * w2_ref[...], axis=1, keepdims=True) + b2_ref[...]


def _make_dec_out():
    blk = 2176
    grid = LP // blk
    return pl.pallas_call(
        _dec_out_body,
        grid=(grid,),
        in_specs=[pl.BlockSpec((blk, H), lambda i: (i, 0)),
                  pl.BlockSpec((1, H), lambda i: (0, 0)),
                  pl.BlockSpec((1, 1), lambda i: (0, 0))],
        out_specs=pl.BlockSpec((blk, 1), lambda i: (i, 0)),
        out_shape=jax.ShapeDtypeStruct((LP, 1), jnp.float32),
    )


_dec_out = _make_dec_out()


def _quad_body(a_ref, b_ref, w1, w2, w3, w4, bias1, bias2,
               o1, o2, o3, o4, *, relu):
    a = a_ref[...]
    b = b_ref[...]
    if relu:
        a = jnp.maximum(a, 0.0)
        b = jnp.maximum(b, 0.0)
    f32 = jnp.float32
    o1[...] = jnp.dot(a, w1[...], preferred_element_type=f32)
    o2[...] = jnp.dot(b, w2[...], preferred_element_type=f32) + bias1[...]
    o3[...] = jnp.dot(b, w3[...], preferred_element_type=f32)
    o4[...] = jnp.dot(a, w4[...], preferred_element_type=f32) + bias2[...]


def _make_quad(relu):
    blk = 1000
    grid = N // blk
    row_spec = pl.BlockSpec((blk, H), lambda i: (i, 0))
    full_spec = pl.BlockSpec((H, H), lambda i: (0, 0))
    bias_spec = pl.BlockSpec((1, H), lambda i: (0, 0))
    return pl.pallas_call(
        functools.partial(_quad_body, relu=relu),
        grid=(grid,),
        in_specs=[row_spec, row_spec, full_spec, full_spec, full_spec,
                  full_spec, bias_spec, bias_spec],
        out_specs=[row_spec, row_spec, row_spec, row_spec],
        out_shape=[jax.ShapeDtypeStruct((N, H), jnp.float32)] * 4,
    )


_quad_plain = _make_quad(relu=False)
_quad_relu = _make_quad(relu=True)


def _dual_body(a_ref, b_ref, w1, w2, bias1, o1, o2):
    f32 = jnp.float32
    o1[...] = jnp.dot(a_ref[...], w1[...], preferred_element_type=f32) + bias1[...]
    o2[...] = jnp.dot(b_ref[...], w2[...], preferred_element_type=f32)


def _make_dual():
    blk = 1000
    grid = N // blk
    row_spec = pl.BlockSpec((blk, H), lambda i: (i, 0))
    full_spec = pl.BlockSpec((H, H), lambda i: (0, 0))
    bias_spec = pl.BlockSpec((1, H), lambda i: (0, 0))
    return pl.pallas_call(
        _dual_body,
        grid=(grid,),
        in_specs=[row_spec, row_spec, full_spec, full_spec, bias_spec],
        out_specs=[row_spec, row_spec],
        out_shape=[jax.ShapeDtypeStruct((N, H), jnp.float32)] * 2,
    )


_dual = _make_dual()


def kernel(x_drug, x_protein, edge_index_drug_protein, edge_index_protein_drug,
           edge_label_index, Wl1_dp, bl1_dp, Wr1_dp, Wl1_pd, bl1_pd, Wr1_pd,
           Wl2_dp, bl2_dp, Wr2_dp, Wl2_pd, bl2_pd, Wr2_pd,
           Wdec1, bdec1, Wdec2, bdec2):
    def pack_edges(ei):
        src = ei[0].reshape(NGRP, 4, CH)
        dst = ei[1].reshape(NGRP, 4, CH)
        return jnp.concatenate([src, dst], axis=1)

    idx_dp = pack_edges(edge_index_drug_protein)
    idx_pd = pack_edges(edge_index_protein_drug)
    lbl = jnp.concatenate(
        [edge_label_index.reshape(-1),
         jnp.zeros((LP - L,), edge_label_index.dtype)])

    b1 = bl1_dp.reshape(1, H)
    b2 = bl1_pd.reshape(1, H)
    b3 = bl2_dp.reshape(1, H)
    b4 = bl2_pd.reshape(1, H)

    tbl_dp, init_prot, tbl_pd, init_drug = _quad_plain(
        x_drug, x_protein, Wl1_dp, Wr1_dp, Wl1_pd, Wr1_pd, b1, b2)
    hpre_prot, hpre_drug = _seg_pair(
        idx_dp, tbl_dp, init_prot, idx_pd, tbl_pd, init_drug)

    tbl2_dp, init2_prot, tbl2_pd, init2_drug = _quad_relu(
        hpre_drug, hpre_prot, Wl2_dp, Wr2_dp, Wl2_pd, Wr2_pd, b3, b4)
    z_prot, z_drug = _seg_pair(
        idx_dp, tbl2_dp, init2_prot, idx_pd, tbl2_pd, init2_drug)

    u_drug, u_prot = _dual(z_drug, z_prot, Wdec1[:H], Wdec1[H:],
                           bdec1.reshape(1, H))
    s = _dec_gather(u_drug, u_prot, lbl)
    out2 = _dec_out(s, Wdec2.reshape(1, H), bdec2.reshape(1, 1))
    out = out2.reshape(-1)[:L]
    return (z_drug, z_prot, out)
